# baseline (speedup 1.0000x reference)
import functools

import jax
import jax.numpy as jnp
from jax import lax
from jax.experimental import pallas as pl
from jax.experimental.pallas import tpu as pltpu
from jax.experimental.pallas import tpu_sc as plsc

D_MODEL = 64
SCALE = 8.0
NB = 4
NOB = 2
BW = 128
T_STEPS = 200
NW = 32


def _body(x2_hbm, table2_hbm, out_hbm, xloc, ridx, gbuf, obuf,
          g0, g1, g2, g3, o0, o1):
  gs = (g0, g1, g2, g3)
  osems = (o0, o1)
  wid = lax.axis_index("s") * 2 + lax.axis_index("c")
  b0 = wid * BW

  pltpu.sync_copy(x2_hbm.at[:, pl.ds(b0, BW)], xloc)

  lanes = lax.iota(jnp.int32, 16)
  rows = [lanes + (j * 16) for j in range(8)]

  def prep(t, slot):
    for j in range(8):
      v = xloc[t, pl.ds(j * 16, 16)]
      ridx[slot, pl.ds(j * 16, 16)] = lax.shift_right_logical(v, 1)

  def start_gather(slot):
    pltpu.async_copy(table2_hbm.at[ridx.at[slot]], gbuf.at[slot], gs[slot])

  def wait_gather(slot):
    pltpu.make_async_copy(
        table2_hbm.at[ridx.at[slot]], gbuf.at[slot], gs[slot]).wait()

  def start_out(t, oslot):
    pltpu.async_copy(
        obuf.at[oslot], out_hbm.at[t, :, pl.ds(b0, BW)], osems[oslot])

  def wait_out(oslot):
    pltpu.make_async_copy(
        obuf.at[oslot], out_hbm.at[0, :, pl.ds(b0, BW)], osems[oslot]).wait()

  def transpose_scale(t, slot, oslot):
    cols0 = []
    for j in range(8):
      v = xloc[t, pl.ds(j * 16, 16)]
      cols0.append((v & 1) * 64)

    def dbody(d0, carry):
      dvec = (d0 + lanes) & (D_MODEL - 1)
      cols = [c + dvec for c in cols0]
      vals = [plsc.load_gather(gbuf.at[slot], [rows[j], cols[j]])
              for j in range(8)]
      for j in range(8):
        plsc.store_scatter(obuf.at[oslot], [dvec, rows[j]], vals[j] * SCALE)
      return carry

    lax.fori_loop(0, D_MODEL, dbody, 0, unroll=4)

  def process(t, b, wait_o):
    slot = b % NB
    oslot = b % NOB
    wait_gather(slot)
    if wait_o:
      wait_out(oslot)
    transpose_scale(t, slot, oslot)
    start_out(t, oslot)

  def fire_next(t_next, b):
    prep(t_next, b % NB)
    start_gather(b % NB)

  for b in range(NB):
    prep(b, b)
    start_gather(b)

  for b in range(NB):
    process(b, b, wait_o=(b >= NOB))
    fire_next(b + NB, b)

  def outer(i, c):
    for b in range(NB):
      t = i * NB + b
      process(t, b, True)
      fire_next(t + NB, b)
    return c
  lax.fori_loop(1, T_STEPS // NB - 1, outer, 0)

  for b in range(NB):
    process(T_STEPS - NB + b, b, True)
  for oslot in range(NOB):
    wait_out(oslot)


@functools.partial(jax.jit, static_argnames=())
def kernel(x, table):
  bsz, tsz = x.shape
  vocab, d = table.shape
  assert d == D_MODEL and tsz == T_STEPS and bsz == NW * BW
  x2 = x.T
  table2 = table.reshape(vocab // 2, 128)

  mesh = plsc.VectorSubcoreMesh(core_axis_name="c", subcore_axis_name="s")
  run = pl.kernel(
      _body,
      mesh=mesh,
      out_type=jax.ShapeDtypeStruct((T_STEPS, D_MODEL, bsz), jnp.float32),
      scratch_types=[
          pltpu.VMEM((T_STEPS, BW), jnp.int32),
          pltpu.VMEM((NB, BW), jnp.int32),
          pltpu.VMEM((NB, BW, 128), jnp.float32),
          pltpu.VMEM((NOB, D_MODEL, BW), jnp.float32),
      ] + [pltpu.SemaphoreType.DMA] * (NB + NOB),
      compiler_params=pltpu.CompilerParams(
          use_tc_tiling_on_sc=True, needs_layout_passes=False,
          disable_bounds_checks=True),
  )
  out_l = run(x2, table2)
  return out_l.transpose(2, 0, 1)


# speedup vs baseline: 1.0957x; 1.0957x over previous
import functools

import jax
import jax.numpy as jnp
from jax import lax
from jax.experimental import pallas as pl
from jax.experimental.pallas import tpu as pltpu
from jax.experimental.pallas import tpu_sc as plsc

D_MODEL = 64
SCALE = 8.0
NB = 4
NOB = 2
BW = 128
T_STEPS = 200
NW = 32


def _body(x2_hbm, table_hbm, out_hbm, xloc, gbuf, obuf,
          g0, g1, g2, g3, o0, o1):
  gs = (g0, g1, g2, g3)
  osems = (o0, o1)
  wid = lax.axis_index("s") * 2 + lax.axis_index("c")
  b0 = wid * BW

  pltpu.sync_copy(x2_hbm.at[:, pl.ds(b0, BW)], xloc)

  lanes = lax.iota(jnp.int32, 16)
  rows = [lanes + (j * 16) for j in range(8)]

  def start_gather(t, slot):
    pltpu.async_copy(table_hbm.at[xloc.at[t]], gbuf.at[slot], gs[slot])

  def wait_gather(slot):
    pltpu.make_async_copy(
        table_hbm.at[xloc.at[0]], gbuf.at[slot], gs[slot]).wait()

  def start_out(t, oslot):
    pltpu.async_copy(obuf.at[oslot], out_hbm.at[t, :, wid], osems[oslot])

  def wait_out(oslot):
    pltpu.make_async_copy(
        obuf.at[oslot], out_hbm.at[0, :, wid], osems[oslot]).wait()

  def transpose_scale(slot, oslot):
    def dbody(d0, carry):
      dvec = (d0 + lanes) & (D_MODEL - 1)
      drv = lax.shift_right_logical(dvec, 3)
      dsv = dvec & 7
      vals = [plsc.load_gather(gbuf.at[slot], [rows[j], dvec])
              for j in range(8)]
      for j in range(8):
        plsc.store_scatter(obuf.at[oslot], [drv, dsv, rows[j]],
                           vals[j] * SCALE)
      return carry

    lax.fori_loop(0, D_MODEL, dbody, 0, unroll=2)

  def process(t, b, wait_o):
    slot = b % NB
    oslot = b % NOB
    wait_gather(slot)
    if wait_o:
      wait_out(oslot)
    transpose_scale(slot, oslot)
    start_out(t, oslot)

  for b in range(NB):
    start_gather(b, b)

  for b in range(NB):
    process(b, b, wait_o=(b >= NOB))
    start_gather(b + NB, b)

  def outer(i, c):
    for b in range(NB):
      t = i * NB + b
      process(t, b, True)
      start_gather(t + NB, b)
    return c
  lax.fori_loop(1, T_STEPS // NB - 1, outer, 0)

  for b in range(NB):
    process(T_STEPS - NB + b, b, True)
  for oslot in range(NOB):
    wait_out(oslot)


@functools.partial(jax.jit, static_argnames=())
def kernel(x, table):
  bsz, tsz = x.shape
  vocab, d = table.shape
  assert d == D_MODEL and tsz == T_STEPS and bsz == NW * BW
  x2 = x.T

  mesh = plsc.VectorSubcoreMesh(core_axis_name="c", subcore_axis_name="s")
  run = pl.kernel(
      _body,
      mesh=mesh,
      out_type=jax.ShapeDtypeStruct((T_STEPS, 8, NW, 8, BW), jnp.float32),
      scratch_types=[
          pltpu.VMEM((T_STEPS, BW), jnp.int32),
          pltpu.VMEM((NB, BW, D_MODEL), jnp.float32),
          pltpu.VMEM((NOB, 8, 8, BW), jnp.float32),
      ] + [pltpu.SemaphoreType.DMA] * (NB + NOB),
      compiler_params=pltpu.CompilerParams(
          use_tc_tiling_on_sc=False, needs_layout_passes=False,
          disable_bounds_checks=True),
  )
  out5 = run(x2, table)
  return out5.transpose(2, 4, 0, 1, 3).reshape(bsz, T_STEPS, D_MODEL)
